# baseline (speedup 1.0000x reference)
import functools
import jax
import jax.numpy as jnp
from jax import lax
from jax.experimental import pallas as pl
from jax.experimental.pallas import tpu as pltpu
from jax.experimental.pallas import tpu_sc as plsc

B, D, N, K, FE = 4, 128, 1024, 16, 4
C_OUT, CE_OUT = 128, 16
BN = 128
NB = N // BN
Q = K * BN
M_ALL = B * N * K
EPS = 1e-5


def _stage1_body(f_ref, idxt_ref, eft_ref, W0_ref, We0_ref,
                 ymax_ref, zmax_ref, efmax_ref, ystats_ref, zstats_ref,
                 g_scr):
    b = pl.program_id(0)
    nb = pl.program_id(1)

    @pl.when(nb == 0)
    def _():
        f = f_ref[0]
        Wa = W0_ref[:, :D]
        Wb = W0_ref[:, D:2 * D]
        g1 = jnp.dot(Wa - Wb, f, preferred_element_type=jnp.float32)
        g2 = jnp.dot(Wb, f, preferred_element_type=jnp.float32)
        g_scr[0] = g1
        g_scr[1] = g2

    g2 = g_scr[1]
    g2_hi = g2.astype(jnp.bfloat16)
    g2_lo = (g2 - g2_hi.astype(jnp.float32)).astype(jnp.bfloat16)

    idxq = idxt_ref[0].reshape(1, Q)
    riota = jax.lax.broadcasted_iota(jnp.int32, (N, Q), 0)
    O = (riota == idxq).astype(jnp.bfloat16)

    y = jnp.dot(g2_hi, O, preferred_element_type=jnp.float32)
    y = y + jnp.dot(g2_lo, O, preferred_element_type=jnp.float32)

    raw = eft_ref[0]
    ef = jnp.concatenate([raw[0], raw[1]], axis=2).reshape(FE, Q)
    We = W0_ref[:, 2 * D:2 * D + FE]
    We0 = We0_ref[...]
    z = jnp.zeros((CE_OUT, Q), jnp.float32)
    for f in range(FE):
        efr = ef[f:f + 1, :]
        y = y + We[:, f:f + 1] * efr
        z = z + We0[:, f:f + 1] * efr

    g1blk = g_scr[0, :, pl.ds(nb * BN, BN)]
    y = y + jnp.concatenate([g1blk] * K, axis=1)

    ysum = jnp.sum(y, axis=1)
    ysq = jnp.sum(y * y, axis=1)
    zsum = jnp.sum(z, axis=1)
    zsq = jnp.sum(z * z, axis=1)

    ymax = y[:, :BN]
    zmaxv = z[:, :BN]
    efmax = ef[:, :BN]
    for j in range(1, K):
        sl = slice(j * BN, (j + 1) * BN)
        ymax = jnp.maximum(ymax, y[:, sl])
        zmaxv = jnp.maximum(zmaxv, z[:, sl])
        efmax = jnp.maximum(efmax, ef[:, sl])
    ymax_ref[0] = ymax
    zmax_ref[0] = zmaxv
    efmax_ref[0] = efmax

    first = jnp.logical_and(b == 0, nb == 0)

    @pl.when(first)
    def _():
        ystats_ref[0, :] = ysum
        ystats_ref[1, :] = ysq
        zstats_ref[0, :] = zsum
        zstats_ref[1, :] = zsq

    @pl.when(jnp.logical_not(first))
    def _():
        ystats_ref[0, :] += ysum
        ystats_ref[1, :] += ysq
        zstats_ref[0, :] += zsum
        zstats_ref[1, :] += zsq


def _stage2_body(f_ref, ymax_ref, zmax_ref, efmax_ref, ystats_ref,
                 zstats_ref, Wsc_ref, out_ref, outef_ref):
    my = ystats_ref[0, :] * (1.0 / M_ALL)
    vy = ystats_ref[1, :] * (1.0 / M_ALL) - my * my
    ry = jax.lax.rsqrt(vy + EPS)
    mz = zstats_ref[0, :] * (1.0 / M_ALL)
    vz = zstats_ref[1, :] * (1.0 / M_ALL) - mz * mz
    rz = jax.lax.rsqrt(vz + EPS)

    fts = jnp.maximum((ymax_ref[...] - my[None, :, None]) * ry[None, :, None], 0.0)
    out_ref[...] = jnp.maximum(f_ref[...] + fts, 0.0)

    Wsc = Wsc_ref[...]
    ssum = jnp.zeros((CE_OUT,), jnp.float32)
    ssq = jnp.zeros((CE_OUT,), jnp.float32)
    s_all = []
    for b in range(B):
        s = jnp.zeros((CE_OUT, N), jnp.float32)
        for f in range(FE):
            s = s + Wsc[:, f:f + 1] * efmax_ref[b, f:f + 1, :]
        s_all.append(s)
        ssum = ssum + jnp.sum(s, axis=1)
        ssq = ssq + jnp.sum(s * s, axis=1)
    ms = ssum * (1.0 / (B * N))
    vs = ssq * (1.0 / (B * N)) - ms * ms
    rs = jax.lax.rsqrt(vs + EPS)
    for b in range(B):
        fts_ef = jnp.maximum((zmax_ref[b] - mz[:, None]) * rz[:, None], 0.0)
        sc = (s_all[b] - ms[:, None]) * rs[:, None]
        outef_ref[b] = jnp.maximum(sc + fts_ef, 0.0)


def _dense_stages(features, idx_t, ef_t, W0, We0, Wsc_ef, interpret=False):
    ymax, zmax, efmax, ystats, zstats = pl.pallas_call(
        _stage1_body,
        grid=(B, NB),
        in_specs=[
            pl.BlockSpec((1, D, N), lambda b, nb: (b, 0, 0)),
            pl.BlockSpec((1, K, BN), lambda b, nb: (b, 0, nb)),
            pl.BlockSpec((1, 2, FE, K, BN // 2), lambda b, nb: (b, nb, 0, 0, 0)),
            pl.BlockSpec((C_OUT, 2 * D + FE), lambda b, nb: (0, 0)),
            pl.BlockSpec((CE_OUT, FE), lambda b, nb: (0, 0)),
        ],
        out_specs=[
            pl.BlockSpec((1, C_OUT, BN), lambda b, nb: (b, 0, nb)),
            pl.BlockSpec((1, CE_OUT, BN), lambda b, nb: (b, 0, nb)),
            pl.BlockSpec((1, FE, BN), lambda b, nb: (b, 0, nb)),
            pl.BlockSpec((2, C_OUT), lambda b, nb: (0, 0)),
            pl.BlockSpec((2, CE_OUT), lambda b, nb: (0, 0)),
        ],
        out_shape=[
            jax.ShapeDtypeStruct((B, C_OUT, N), jnp.float32),
            jax.ShapeDtypeStruct((B, CE_OUT, N), jnp.float32),
            jax.ShapeDtypeStruct((B, FE, N), jnp.float32),
            jax.ShapeDtypeStruct((2, C_OUT), jnp.float32),
            jax.ShapeDtypeStruct((2, CE_OUT), jnp.float32),
        ],
        scratch_shapes=[pltpu.VMEM((2, D, N), jnp.float32)],
        interpret=interpret,
    )(features, idx_t, ef_t, W0, We0)

    out, out_ef = pl.pallas_call(
        _stage2_body,
        out_shape=[
            jax.ShapeDtypeStruct((B, C_OUT, N), jnp.float32),
            jax.ShapeDtypeStruct((B, CE_OUT, N), jnp.float32),
        ],
        interpret=interpret,
    )(features, ymax, zmax, efmax, ystats, zstats, Wsc_ef)
    return out, out_ef


P = 16384
NQUART = 4
QW = N // NQUART
ROWS_T = N // 16
CE = 2048
NCH = P // CE


def _sc_ef_body(el_ref, eft_ref, idx_ref, ef_ref,
                idxb, srcb, dstb, valb, dens, efblk, sem0, sem1):
    c = lax.axis_index("c")
    s = lax.axis_index("s")
    iota = lax.broadcasted_iota(jnp.int32, (16,), 0)
    zero16 = jnp.zeros((16,), jnp.float32)
    sems = (sem0, sem1)

    for bb in range(2):
        b = c * 2 + bb
        n0 = s * ROWS_T
        pltpu.sync_copy(idx_ref.at[b, pl.ds(n0, ROWS_T)], idxb)

        def zeroblk(i, _):
            plsc.store_scatter(efblk, [iota * 0 + i // (K * ROWS_T // 16),
                                       (i * 16 + iota) // ROWS_T % K,
                                       (i * 16 + iota) % ROWS_T],
                               zero16)
            return 0
        lax.fori_loop(0, FE * K * ROWS_T // 16, zeroblk, 0)

        def quarter(q, _):
            lo = q * QW

            def qzero(r, _):
                dv = idxb[r, :]
                m = jnp.logical_and(dv >= lo, dv < lo + QW)
                addr = (r * QW + dv - lo) * FE
                for f in range(FE):
                    plsc.store_scatter(dens, [addr + f], zero16, mask=m)
                return 0
            lax.fori_loop(0, ROWS_T, qzero, 0)

            def scan_chunk(base, sb, db, vb):
                def step(i, _):
                    sv = sb[pl.ds(i * 16, 16)]
                    dv = db[pl.ds(i * 16, 16)]
                    m = jnp.logical_and(
                        jnp.logical_and(sv >= n0, sv < n0 + ROWS_T),
                        jnp.logical_and(dv >= lo, dv < lo + QW))
                    nmatch = jnp.sum(m.astype(jnp.int32))

                    @pl.when(nmatch > 0)
                    def _():
                        addr = ((sv - n0) * QW + dv - lo) * FE
                        rid = i * 16 + iota
                        for f in range(FE):
                            vals = plsc.load_gather(vb, [rid, iota * 0 + f],
                                                    mask=m)
                            plsc.addupdate_scatter(dens, [addr + f], vals,
                                                   mask=m)
                    return 0
                lax.fori_loop(0, CE // 16, step, 0)

            def start(ch, buf):
                off = ch * CE
                return (pltpu.async_copy(el_ref.at[b, 0, pl.ds(off, CE)],
                                         srcb.at[buf], sems[buf]),
                        pltpu.async_copy(el_ref.at[b, 1, pl.ds(off, CE)],
                                         dstb.at[buf], sems[buf]),
                        pltpu.async_copy(eft_ref.at[b, pl.ds(off, CE)],
                                         valb.at[buf], sems[buf]))

            cps = start(0, 0)
            for ch in range(NCH):
                buf = ch % 2
                for cp in cps:
                    cp.wait()
                if ch + 1 < NCH:
                    cps = start(ch + 1, 1 - buf)
                scan_chunk(ch * CE, srcb.at[buf], dstb.at[buf], valb.at[buf])

            def qgather(r, _):
                dv = idxb[r, :]
                m = jnp.logical_and(dv >= lo, dv < lo + QW)
                addr = (r * QW + dv - lo) * FE
                for f in range(FE):
                    g = plsc.load_gather(dens, [addr + f], mask=m)
                    plsc.store_scatter(efblk, [iota * 0 + f, iota,
                                               iota * 0 + r], g, mask=m)
                return 0
            lax.fori_loop(0, ROWS_T, qgather, 0)
            return 0

        lax.fori_loop(0, NQUART, quarter, 0)
        pltpu.sync_copy(efblk, ef_ref.at[b, s])


def _ef_sparsecore(edge_list, edge_features, idx):
    eft = jnp.transpose(edge_features, (0, 2, 1))
    mesh = plsc.VectorSubcoreMesh(core_axis_name="c", subcore_axis_name="s")
    kern = functools.partial(
        pl.kernel,
        mesh=mesh,
        compiler_params=pltpu.CompilerParams(use_tc_tiling_on_sc=False,
                                             needs_layout_passes=False),
        out_type=jax.ShapeDtypeStruct((B, N // ROWS_T, FE, K, ROWS_T),
                                      jnp.float32),
        scratch_types=[
            pltpu.VMEM((ROWS_T, K), jnp.int32),
            pltpu.VMEM((2, CE), jnp.int32),
            pltpu.VMEM((2, CE), jnp.int32),
            pltpu.VMEM((2, CE, FE), jnp.float32),
            pltpu.VMEM((ROWS_T * QW * FE,), jnp.float32),
            pltpu.VMEM((FE, K, ROWS_T), jnp.float32),
            pltpu.SemaphoreType.DMA,
            pltpu.SemaphoreType.DMA,
        ],
    )(_sc_ef_body)
    return kern(edge_list, eft, idx)


def _ef_xla(edge_list, edge_features, idx):
    src = edge_list[:, 0, :]
    dst = edge_list[:, 1, :]
    key_e = src * N + dst
    eft = jnp.transpose(edge_features, (0, 2, 1))
    dense = jnp.zeros((B, N * N, FE), jnp.float32)
    dense = jax.vmap(lambda d, k, v: d.at[k].add(v))(dense, key_e, eft)
    key_q = jnp.arange(N)[None, :, None] * N + idx
    ef = jax.vmap(lambda d, k: d[k.reshape(-1)])(dense, key_q)
    ef = ef.reshape(B, N // 64, 64, K, FE)
    return jnp.transpose(ef, (0, 1, 4, 3, 2))


@jax.jit
def kernel(points, features, edge_list, edge_features, idx, W0, We0, Wsc_ef):
    del points
    ef_t = _ef_sparsecore(edge_list, edge_features, idx)
    idx_t = jnp.transpose(idx, (0, 2, 1))
    return _dense_stages(features, idx_t, ef_t, W0, We0, Wsc_ef)


# speedup vs baseline: 1.3948x; 1.3948x over previous
import functools
import jax
import jax.numpy as jnp
from jax import lax
from jax.experimental import pallas as pl
from jax.experimental.pallas import tpu as pltpu
from jax.experimental.pallas import tpu_sc as plsc

B, D, N, K, FE = 4, 128, 1024, 16, 4
C_OUT, CE_OUT = 128, 16
BN = 128
NB = N // BN
Q = K * BN
M_ALL = B * N * K
EPS = 1e-5


def _stage1_body(f_ref, idxt_ref, eft_ref, W0_ref, We0_ref,
                 ymax_ref, zmax_ref, efmax_ref, ystats_ref, zstats_ref,
                 g_scr):
    b = pl.program_id(0)
    nb = pl.program_id(1)

    @pl.when(nb == 0)
    def _():
        f = f_ref[0]
        Wa = W0_ref[:, :D]
        Wb = W0_ref[:, D:2 * D]
        g1 = jnp.dot(Wa - Wb, f, preferred_element_type=jnp.float32)
        g2 = jnp.dot(Wb, f, preferred_element_type=jnp.float32)
        g_scr[0] = g1
        g_scr[1] = g2

    g2 = g_scr[1]
    g2_hi = g2.astype(jnp.bfloat16)
    g2_lo = (g2 - g2_hi.astype(jnp.float32)).astype(jnp.bfloat16)

    idxq = idxt_ref[0].reshape(1, Q)
    riota = jax.lax.broadcasted_iota(jnp.int32, (N, Q), 0)
    O = (riota == idxq).astype(jnp.bfloat16)

    y = jnp.dot(g2_hi, O, preferred_element_type=jnp.float32)
    y = y + jnp.dot(g2_lo, O, preferred_element_type=jnp.float32)

    raw = eft_ref[0]
    ef = jnp.concatenate([raw[0], raw[1]], axis=2).reshape(FE, Q)
    We = W0_ref[:, 2 * D:2 * D + FE]
    We0 = We0_ref[...]
    z = jnp.zeros((CE_OUT, Q), jnp.float32)
    for f in range(FE):
        efr = ef[f:f + 1, :]
        y = y + We[:, f:f + 1] * efr
        z = z + We0[:, f:f + 1] * efr

    g1blk = g_scr[0, :, pl.ds(nb * BN, BN)]
    y = y + jnp.concatenate([g1blk] * K, axis=1)

    ysum = jnp.sum(y, axis=1)
    ysq = jnp.sum(y * y, axis=1)
    zsum = jnp.sum(z, axis=1)
    zsq = jnp.sum(z * z, axis=1)

    ymax = y[:, :BN]
    zmaxv = z[:, :BN]
    efmax = ef[:, :BN]
    for j in range(1, K):
        sl = slice(j * BN, (j + 1) * BN)
        ymax = jnp.maximum(ymax, y[:, sl])
        zmaxv = jnp.maximum(zmaxv, z[:, sl])
        efmax = jnp.maximum(efmax, ef[:, sl])
    ymax_ref[0] = ymax
    zmax_ref[0] = zmaxv
    efmax_ref[0] = efmax

    first = jnp.logical_and(b == 0, nb == 0)

    @pl.when(first)
    def _():
        ystats_ref[0, :] = ysum
        ystats_ref[1, :] = ysq
        zstats_ref[0, :] = zsum
        zstats_ref[1, :] = zsq

    @pl.when(jnp.logical_not(first))
    def _():
        ystats_ref[0, :] += ysum
        ystats_ref[1, :] += ysq
        zstats_ref[0, :] += zsum
        zstats_ref[1, :] += zsq


def _stage2_body(f_ref, ymax_ref, zmax_ref, efmax_ref, ystats_ref,
                 zstats_ref, Wsc_ref, out_ref, outef_ref):
    my = ystats_ref[0, :] * (1.0 / M_ALL)
    vy = ystats_ref[1, :] * (1.0 / M_ALL) - my * my
    ry = jax.lax.rsqrt(vy + EPS)
    mz = zstats_ref[0, :] * (1.0 / M_ALL)
    vz = zstats_ref[1, :] * (1.0 / M_ALL) - mz * mz
    rz = jax.lax.rsqrt(vz + EPS)

    fts = jnp.maximum((ymax_ref[...] - my[None, :, None]) * ry[None, :, None], 0.0)
    out_ref[...] = jnp.maximum(f_ref[...] + fts, 0.0)

    Wsc = Wsc_ref[...]
    ssum = jnp.zeros((CE_OUT,), jnp.float32)
    ssq = jnp.zeros((CE_OUT,), jnp.float32)
    s_all = []
    for b in range(B):
        s = jnp.zeros((CE_OUT, N), jnp.float32)
        for f in range(FE):
            s = s + Wsc[:, f:f + 1] * efmax_ref[b, f:f + 1, :]
        s_all.append(s)
        ssum = ssum + jnp.sum(s, axis=1)
        ssq = ssq + jnp.sum(s * s, axis=1)
    ms = ssum * (1.0 / (B * N))
    vs = ssq * (1.0 / (B * N)) - ms * ms
    rs = jax.lax.rsqrt(vs + EPS)
    for b in range(B):
        fts_ef = jnp.maximum((zmax_ref[b] - mz[:, None]) * rz[:, None], 0.0)
        sc = (s_all[b] - ms[:, None]) * rs[:, None]
        outef_ref[b] = jnp.maximum(sc + fts_ef, 0.0)


def _dense_stages(features, idx_t, ef_t, W0, We0, Wsc_ef, interpret=False):
    ymax, zmax, efmax, ystats, zstats = pl.pallas_call(
        _stage1_body,
        grid=(B, NB),
        in_specs=[
            pl.BlockSpec((1, D, N), lambda b, nb: (b, 0, 0)),
            pl.BlockSpec((1, K, BN), lambda b, nb: (b, 0, nb)),
            pl.BlockSpec((1, 2, FE, K, BN // 2), lambda b, nb: (b, nb, 0, 0, 0)),
            pl.BlockSpec((C_OUT, 2 * D + FE), lambda b, nb: (0, 0)),
            pl.BlockSpec((CE_OUT, FE), lambda b, nb: (0, 0)),
        ],
        out_specs=[
            pl.BlockSpec((1, C_OUT, BN), lambda b, nb: (b, 0, nb)),
            pl.BlockSpec((1, CE_OUT, BN), lambda b, nb: (b, 0, nb)),
            pl.BlockSpec((1, FE, BN), lambda b, nb: (b, 0, nb)),
            pl.BlockSpec((2, C_OUT), lambda b, nb: (0, 0)),
            pl.BlockSpec((2, CE_OUT), lambda b, nb: (0, 0)),
        ],
        out_shape=[
            jax.ShapeDtypeStruct((B, C_OUT, N), jnp.float32),
            jax.ShapeDtypeStruct((B, CE_OUT, N), jnp.float32),
            jax.ShapeDtypeStruct((B, FE, N), jnp.float32),
            jax.ShapeDtypeStruct((2, C_OUT), jnp.float32),
            jax.ShapeDtypeStruct((2, CE_OUT), jnp.float32),
        ],
        scratch_shapes=[pltpu.VMEM((2, D, N), jnp.float32)],
        interpret=interpret,
    )(features, idx_t, ef_t, W0, We0)

    out, out_ef = pl.pallas_call(
        _stage2_body,
        out_shape=[
            jax.ShapeDtypeStruct((B, C_OUT, N), jnp.float32),
            jax.ShapeDtypeStruct((B, CE_OUT, N), jnp.float32),
        ],
        interpret=interpret,
    )(features, ymax, zmax, efmax, ystats, zstats, Wsc_ef)
    return out, out_ef


P = 16384
NQUART = 4
QW = N // NQUART
ROWS_T = N // 16
CE = 2048
NCH = P // CE


def _sc_ef_body(el_ref, eft_ref, idx_ref, ef_ref,
                idxb, srcb, dstb, valb, dens, efblk, sem0, sem1):
    c = lax.axis_index("c")
    s = lax.axis_index("s")
    iota = lax.broadcasted_iota(jnp.int32, (16,), 0)
    zero16 = jnp.zeros((16,), jnp.float32)
    sems = (sem0, sem1)

    for bb in range(2):
        b = c * 2 + bb
        n0 = s * ROWS_T
        pltpu.sync_copy(idx_ref.at[b, pl.ds(n0, ROWS_T)], idxb)

        def zeroblk(i, _):
            plsc.store_scatter(efblk, [iota * 0 + i // (K * ROWS_T // 16),
                                       (i * 16 + iota) // ROWS_T % K,
                                       (i * 16 + iota) % ROWS_T],
                               zero16)
            return 0
        lax.fori_loop(0, FE * K * ROWS_T // 16, zeroblk, 0)

        def quarter(q, _):
            lo = q * QW

            def qzero(r, _):
                dv = idxb[r, :]
                m = jnp.logical_and(dv >= lo, dv < lo + QW)
                addr = (r * QW + dv - lo) * FE
                for f in range(FE):
                    plsc.store_scatter(dens, [addr + f], zero16, mask=m)
                return 0
            lax.fori_loop(0, ROWS_T, qzero, 0)

            def scan_chunk(base, sb, db, vb):
                def step(i, _):
                    sv = sb[pl.ds(i * 16, 16)]
                    dv = db[pl.ds(i * 16, 16)]
                    m = jnp.logical_and(
                        jnp.logical_and(sv >= n0, sv < n0 + ROWS_T),
                        jnp.logical_and(dv >= lo, dv < lo + QW))
                    addr = ((sv - n0) * QW + dv - lo) * FE
                    rid = i * 16 + iota
                    for f in range(FE):
                        vals = plsc.load_gather(vb, [rid, iota * 0 + f],
                                                mask=m)
                        plsc.addupdate_scatter(dens, [addr + f], vals,
                                               mask=m)
                    return 0
                lax.fori_loop(0, CE // 16, step, 0, unroll=4)

            def start(ch, buf):
                off = ch * CE
                return (pltpu.async_copy(el_ref.at[b, 0, pl.ds(off, CE)],
                                         srcb.at[buf], sems[buf]),
                        pltpu.async_copy(el_ref.at[b, 1, pl.ds(off, CE)],
                                         dstb.at[buf], sems[buf]),
                        pltpu.async_copy(eft_ref.at[b, pl.ds(off, CE)],
                                         valb.at[buf], sems[buf]))

            cps = start(0, 0)
            for ch in range(NCH):
                buf = ch % 2
                for cp in cps:
                    cp.wait()
                if ch + 1 < NCH:
                    cps = start(ch + 1, 1 - buf)
                scan_chunk(ch * CE, srcb.at[buf], dstb.at[buf], valb.at[buf])

            def qgather(r, _):
                dv = idxb[r, :]
                m = jnp.logical_and(dv >= lo, dv < lo + QW)
                addr = (r * QW + dv - lo) * FE
                for f in range(FE):
                    g = plsc.load_gather(dens, [addr + f], mask=m)
                    plsc.store_scatter(efblk, [iota * 0 + f, iota,
                                               iota * 0 + r], g, mask=m)
                return 0
            lax.fori_loop(0, ROWS_T, qgather, 0)
            return 0

        lax.fori_loop(0, NQUART, quarter, 0)
        pltpu.sync_copy(efblk, ef_ref.at[b, s])


def _ef_sparsecore(edge_list, edge_features, idx):
    eft = jnp.transpose(edge_features, (0, 2, 1))
    mesh = plsc.VectorSubcoreMesh(core_axis_name="c", subcore_axis_name="s")
    kern = functools.partial(
        pl.kernel,
        mesh=mesh,
        compiler_params=pltpu.CompilerParams(use_tc_tiling_on_sc=False,
                                             needs_layout_passes=False),
        out_type=jax.ShapeDtypeStruct((B, N // ROWS_T, FE, K, ROWS_T),
                                      jnp.float32),
        scratch_types=[
            pltpu.VMEM((ROWS_T, K), jnp.int32),
            pltpu.VMEM((2, CE), jnp.int32),
            pltpu.VMEM((2, CE), jnp.int32),
            pltpu.VMEM((2, CE, FE), jnp.float32),
            pltpu.VMEM((ROWS_T * QW * FE,), jnp.float32),
            pltpu.VMEM((FE, K, ROWS_T), jnp.float32),
            pltpu.SemaphoreType.DMA,
            pltpu.SemaphoreType.DMA,
        ],
    )(_sc_ef_body)
    return kern(edge_list, eft, idx)


def _ef_xla(edge_list, edge_features, idx):
    src = edge_list[:, 0, :]
    dst = edge_list[:, 1, :]
    key_e = src * N + dst
    eft = jnp.transpose(edge_features, (0, 2, 1))
    dense = jnp.zeros((B, N * N, FE), jnp.float32)
    dense = jax.vmap(lambda d, k, v: d.at[k].add(v))(dense, key_e, eft)
    key_q = jnp.arange(N)[None, :, None] * N + idx
    ef = jax.vmap(lambda d, k: d[k.reshape(-1)])(dense, key_q)
    ef = ef.reshape(B, N // 64, 64, K, FE)
    return jnp.transpose(ef, (0, 1, 4, 3, 2))


@jax.jit
def kernel(points, features, edge_list, edge_features, idx, W0, We0, Wsc_ef):
    del points
    ef_t = _ef_sparsecore(edge_list, edge_features, idx)
    idx_t = jnp.transpose(idx, (0, 2, 1))
    return _dense_stages(features, idx_t, ef_t, W0, We0, Wsc_ef)
